# baseline (speedup 1.0000x reference)
import functools

import jax
import jax.numpy as jnp
from jax import lax
from jax.experimental import pallas as pl
from jax.experimental.pallas import tpu as pltpu
from jax.experimental.pallas import tpu_sc as plsc

BATCH = 16384
VOCAB = 100000
EMBED = 64
NUM_TABLES = 3
NW = 32
BPW = BATCH // NW

_PACK_COLS = 512
_PACK_GRID = -(-VOCAB // _PACK_COLS)

_mesh = plsc.VectorSubcoreMesh(core_axis_name="c", subcore_axis_name="s")


def _pack_body(in_ref, out_ref):
    x = in_ref[...]
    y = x.reshape(EMBED, _PACK_COLS // 2, 2)
    a = y[:, :, 0].T
    b = y[:, :, 1].T
    out_ref[...] = jnp.concatenate([a, b], axis=1)


def _pack(tT):
    return pl.pallas_call(
        _pack_body,
        grid=(_PACK_GRID,),
        in_specs=[pl.BlockSpec((EMBED, _PACK_COLS), lambda j: (0, j))],
        out_specs=pl.BlockSpec((_PACK_COLS // 2, 2 * EMBED), lambda j: (j, 0)),
        out_shape=jax.ShapeDtypeStruct((VOCAB // 2, 2 * EMBED), jnp.float32),
    )(tT)


@functools.partial(
    pl.kernel,
    mesh=_mesh,
    compiler_params=pltpu.CompilerParams(use_tc_tiling_on_sc=False),
    out_type=jax.ShapeDtypeStruct((BATCH, EMBED), jnp.float32),
    scratch_types=[
        pltpu.VMEM((BPW,), jnp.int32),
        pltpu.VMEM((BPW, EMBED), jnp.float32),
        pltpu.SemaphoreType.DMA,
    ],
)
def _gather_one(idx_hbm, table_hbm, out_hbm, idx_v, rows_v, sem):
    wid = lax.axis_index("s") * 2 + lax.axis_index("c")
    base = wid * BPW
    pltpu.sync_copy(idx_hbm.at[pl.ds(base, BPW)], idx_v)
    pltpu.async_copy(table_hbm.at[idx_v], rows_v, sem).wait()
    pltpu.sync_copy(rows_v, out_hbm.at[pl.ds(base, BPW)])


def kernel(x, emb_mi, emb_mo, emb_mtext):
    xT = jnp.transpose(x)
    outs = []
    for t, emb in enumerate((emb_mi, emb_mo, emb_mtext)):
        packed = _pack(jnp.transpose(emb))
        table = packed.reshape(VOCAB, EMBED)
        outs.append(_gather_one(xT[t], table))
    return jnp.concatenate(outs, axis=1)


# speedup vs baseline: 9.3728x; 9.3728x over previous
import functools

import jax
import jax.numpy as jnp
from jax import lax
from jax.experimental import pallas as pl
from jax.experimental.pallas import tpu as pltpu
from jax.experimental.pallas import tpu_sc as plsc

BATCH = 16384
VOCAB = 100000
EMBED = 64
NUM_TABLES = 3
NW = 32
BPW = BATCH // NW

_TC = 512
_TGRID = -(-VOCAB // _TC)

_mesh = plsc.VectorSubcoreMesh(core_axis_name="c", subcore_axis_name="s")


def _transpose_body(in_ref, out_ref):
    xt = in_ref[...].T
    out_ref[...] = jnp.concatenate([xt, xt], axis=1)


def _to_rows(tT):
    return pl.pallas_call(
        _transpose_body,
        grid=(_TGRID,),
        in_specs=[pl.BlockSpec((EMBED, _TC), lambda j: (0, j))],
        out_specs=pl.BlockSpec((_TC, 2 * EMBED), lambda j: (j, 0)),
        out_shape=jax.ShapeDtypeStruct((VOCAB, 2 * EMBED), jnp.float32),
    )(tT)


@functools.partial(
    pl.kernel,
    mesh=_mesh,
    out_type=jax.ShapeDtypeStruct((BATCH, 2 * EMBED), jnp.float32),
    scratch_types=[
        pltpu.VMEM((BPW,), jnp.int32),
        pltpu.VMEM((BPW, 2 * EMBED), jnp.float32),
        pltpu.SemaphoreType.DMA,
    ],
)
def _gather_one(idx_hbm, table_hbm, out_hbm, idx_v, rows_v, sem):
    wid = lax.axis_index("s") * 2 + lax.axis_index("c")
    base = wid * BPW
    pltpu.sync_copy(idx_hbm.at[pl.ds(base, BPW)], idx_v)
    pltpu.async_copy(table_hbm.at[idx_v], rows_v, sem).wait()
    pltpu.sync_copy(rows_v, out_hbm.at[pl.ds(base, BPW)])


def kernel(x, emb_mi, emb_mo, emb_mtext):
    xT = jnp.transpose(x)
    outs = []
    for t, emb in enumerate((emb_mi, emb_mo, emb_mtext)):
        rows = _to_rows(jnp.transpose(emb))
        outs.append(_gather_one(xT[t], rows))
    return jnp.concatenate([o[:, :EMBED] for o in outs], axis=1)


# speedup vs baseline: 22.6247x; 2.4139x over previous
import functools

import jax
import jax.numpy as jnp
from jax import lax
from jax.experimental import pallas as pl
from jax.experimental.pallas import tpu as pltpu
from jax.experimental.pallas import tpu_sc as plsc

BATCH = 16384
VOCAB = 100000
EMBED = 64
NUM_TABLES = 3
NW = 32
BPW = BATCH // NW

_TC = 4096
_TGRID = -(-VOCAB // _TC)

_mesh = plsc.VectorSubcoreMesh(core_axis_name="c", subcore_axis_name="s")


def _transpose_body(in_ref, out_ref):
    xt = in_ref[...].T
    out_ref[...] = jnp.concatenate([xt, xt], axis=1)


def _to_rows(tT):
    return pl.pallas_call(
        _transpose_body,
        grid=(_TGRID,),
        in_specs=[pl.BlockSpec((EMBED, _TC), lambda j: (0, j))],
        out_specs=pl.BlockSpec((_TC, 2 * EMBED), lambda j: (j, 0)),
        out_shape=jax.ShapeDtypeStruct((VOCAB, 2 * EMBED), jnp.float32),
    )(tT)


@functools.partial(
    pl.kernel,
    mesh=_mesh,
    out_type=jax.ShapeDtypeStruct((BATCH, 2 * EMBED), jnp.float32),
    scratch_types=[
        pltpu.VMEM((BPW,), jnp.int32),
        pltpu.VMEM((BPW, 2 * EMBED), jnp.float32),
        pltpu.SemaphoreType.DMA,
    ],
)
def _gather_one(idx_hbm, table_hbm, out_hbm, idx_v, rows_v, sem):
    wid = lax.axis_index("s") * 2 + lax.axis_index("c")
    base = wid * BPW
    pltpu.sync_copy(idx_hbm.at[pl.ds(base, BPW)], idx_v)
    pltpu.async_copy(table_hbm.at[idx_v], rows_v, sem).wait()
    pltpu.sync_copy(rows_v, out_hbm.at[pl.ds(base, BPW)])


def kernel(x, emb_mi, emb_mo, emb_mtext):
    xT = jnp.transpose(x)
    outs = []
    for t, emb in enumerate((emb_mi, emb_mo, emb_mtext)):
        rows = _to_rows(jnp.transpose(emb))
        outs.append(_gather_one(xT[t], rows))
    return jnp.concatenate([o[:, :EMBED] for o in outs], axis=1)


# speedup vs baseline: 25.6902x; 1.1355x over previous
import functools

import jax
import jax.numpy as jnp
from jax import lax
from jax.experimental import pallas as pl
from jax.experimental.pallas import tpu as pltpu
from jax.experimental.pallas import tpu_sc as plsc

BATCH = 16384
VOCAB = 100000
EMBED = 64
NW = 32
BPW = BATCH // NW

_TC = 8192
_TGRID = -(-VOCAB // _TC)

_mesh = plsc.VectorSubcoreMesh(core_axis_name="c", subcore_axis_name="s")


def _pack2_body(a_ref, b_ref, out_ref):
    out_ref[...] = jnp.concatenate([a_ref[...].T, b_ref[...].T], axis=1)


def _pack2(tTa, tTb):
    return pl.pallas_call(
        _pack2_body,
        grid=(_TGRID,),
        in_specs=[
            pl.BlockSpec((EMBED, _TC), lambda j: (0, j)),
            pl.BlockSpec((EMBED, _TC), lambda j: (0, j)),
        ],
        out_specs=pl.BlockSpec((_TC, 2 * EMBED), lambda j: (j, 0)),
        out_shape=jax.ShapeDtypeStruct((VOCAB, 2 * EMBED), jnp.float32),
    )(tTa, tTb)


@functools.partial(
    pl.kernel,
    mesh=_mesh,
    out_type=jax.ShapeDtypeStruct((BATCH, 2 * EMBED), jnp.float32),
    scratch_types=[
        pltpu.VMEM((BPW,), jnp.int32),
        pltpu.VMEM((BPW, 2 * EMBED), jnp.float32),
        pltpu.SemaphoreType.DMA,
    ],
)
def _gather_one(idx_hbm, table_hbm, out_hbm, idx_v, rows_v, sem):
    wid = lax.axis_index("s") * 2 + lax.axis_index("c")
    base = wid * BPW
    pltpu.sync_copy(idx_hbm.at[pl.ds(base, BPW)], idx_v)
    pltpu.async_copy(table_hbm.at[idx_v], rows_v, sem).wait()
    pltpu.sync_copy(rows_v, out_hbm.at[pl.ds(base, BPW)])


def kernel(x, emb_mi, emb_mo, emb_mtext):
    xT = jnp.transpose(x)
    rows01 = _pack2(jnp.transpose(emb_mi), jnp.transpose(emb_mo))
    o0 = _gather_one(xT[0], rows01)
    o1 = _gather_one(xT[1], rows01)
    t2 = jnp.transpose(emb_mtext)
    rows22 = _pack2(t2, t2)
    o2 = _gather_one(xT[2], rows22)
    return jnp.concatenate(
        [o0[:, :EMBED], o1[:, EMBED:], o2[:, :EMBED]], axis=1)


# speedup vs baseline: 27.9728x; 1.0889x over previous
import functools

import jax
import jax.numpy as jnp
from jax import lax
from jax.experimental import pallas as pl
from jax.experimental.pallas import tpu as pltpu
from jax.experimental.pallas import tpu_sc as plsc

BATCH = 16384
VOCAB = 100000
EMBED = 64
NW = 32
BPW = BATCH // NW

_TC = 16384
_TGRID = -(-VOCAB // _TC)

_mesh = plsc.VectorSubcoreMesh(core_axis_name="c", subcore_axis_name="s")


def _pack2_body(a_ref, b_ref, out_ref):
    out_ref[...] = jnp.concatenate([a_ref[...].T, b_ref[...].T], axis=1)


def _pack_dup_body(a_ref, out_ref):
    at = a_ref[...].T
    out_ref[...] = jnp.concatenate([at, at], axis=1)


def _pack_dup(tTa):
    return pl.pallas_call(
        _pack_dup_body,
        grid=(_TGRID,),
        in_specs=[pl.BlockSpec((EMBED, _TC), lambda j: (0, j))],
        out_specs=pl.BlockSpec((_TC, 2 * EMBED), lambda j: (j, 0)),
        out_shape=jax.ShapeDtypeStruct((VOCAB, 2 * EMBED), jnp.float32),
    )(tTa)


def _pack2(tTa, tTb):
    return pl.pallas_call(
        _pack2_body,
        grid=(_TGRID,),
        in_specs=[
            pl.BlockSpec((EMBED, _TC), lambda j: (0, j)),
            pl.BlockSpec((EMBED, _TC), lambda j: (0, j)),
        ],
        out_specs=pl.BlockSpec((_TC, 2 * EMBED), lambda j: (j, 0)),
        out_shape=jax.ShapeDtypeStruct((VOCAB, 2 * EMBED), jnp.float32),
    )(tTa, tTb)


@functools.partial(
    pl.kernel,
    mesh=_mesh,
    out_type=jax.ShapeDtypeStruct((BATCH, 2 * EMBED), jnp.float32),
    scratch_types=[
        pltpu.VMEM((BPW,), jnp.int32),
        pltpu.VMEM((BPW, 2 * EMBED), jnp.float32),
        pltpu.SemaphoreType.DMA,
    ],
)
def _gather_one(idx_hbm, table_hbm, out_hbm, idx_v, rows_v, sem):
    wid = lax.axis_index("s") * 2 + lax.axis_index("c")
    base = wid * BPW
    pltpu.sync_copy(idx_hbm.at[pl.ds(base, BPW)], idx_v)
    pltpu.async_copy(table_hbm.at[idx_v], rows_v, sem).wait()
    pltpu.sync_copy(rows_v, out_hbm.at[pl.ds(base, BPW)])


def kernel(x, emb_mi, emb_mo, emb_mtext):
    xT = jnp.transpose(x)
    rows01 = _pack2(jnp.transpose(emb_mi), jnp.transpose(emb_mo))
    o0 = _gather_one(xT[0], rows01)
    o1 = _gather_one(xT[1], rows01)
    rows22 = _pack_dup(jnp.transpose(emb_mtext))
    o2 = _gather_one(xT[2], rows22)
    return jnp.concatenate(
        [o0[:, :EMBED], o1[:, EMBED:], o2[:, :EMBED]], axis=1)
